# Initial kernel scaffold; baseline (speedup 1.0000x reference)
import jax
import jax.numpy as jnp
from jax.experimental import pallas as pl


def kernel(x_lhand, x_rhand, j_lhand, j_rhand, m_contact, x_obj, point_cloud, fc_lw, fc_lb, fc_rw, fc_rb, out_lw, out_lb, out_rw, out_rb, Wqkv, bqkv, Wo, bo, W1, b1f, W2, b2f, ln1_g, ln1_b, ln2_g, ln2_b):
    raise NotImplementedError("write your pallas kernel here")


# speedup vs baseline: 1.0212x; 1.0212x over previous
import numpy as np
import jax
import jax.numpy as jnp
from jax.experimental import pallas as pl
from jax.experimental.pallas import tpu as pltpu

B, L, N = 16, 64, 1024
HAND, J, HID = 99, 21, 512
NL, NH, DH = 4, 8, 64
F32 = jnp.float32


def _pe_table(n, d):
    pos = np.arange(n)[:, None].astype(np.float32)
    div = np.exp(np.arange(0, d, 2).astype(np.float32) * (-np.log(10000.0) / d))
    pe = np.zeros((n, d), np.float32)
    pe[:, 0::2] = np.sin(pos * div)
    pe[:, 1::2] = np.cos(pos * div)
    return pe


def _geom_body(xobj, jxl, jyl, jzl, jxr, jyr, jzr, px, py, pz, mc, xhl, xhr,
               wsl, wsr, wmc, wpcn, bl, br, ol, orr):
    xo = xobj[0]
    tx, ty, tz = xo[:, 0:1], xo[:, 1:2], xo[:, 2:3]
    a1x, a1y, a1z = xo[:, 3:4], xo[:, 4:5], xo[:, 5:6]
    a2x, a2y, a2z = xo[:, 6:7], xo[:, 7:8], xo[:, 8:9]
    n1 = jax.lax.rsqrt(a1x * a1x + a1y * a1y + a1z * a1z)
    b1x, b1y, b1z = a1x * n1, a1y * n1, a1z * n1
    dd = b1x * a2x + b1y * a2y + b1z * a2z
    c2x, c2y, c2z = a2x - dd * b1x, a2y - dd * b1y, a2z - dd * b1z
    n2 = jax.lax.rsqrt(c2x * c2x + c2y * c2y + c2z * c2z)
    b2x, b2y, b2z = c2x * n2, c2y * n2, c2z * n2
    b3x = b1y * b2z - b1z * b2y
    b3y = b1z * b2x - b1x * b2z
    b3z = b1x * b2y - b1y * b2x

    bf = lambda v: v.astype(jnp.bfloat16).astype(F32)
    px3, py3, pz3 = bf(px[...]), bf(py[...]), bf(pz[...])
    e = lambda v: bf(v)[:, :, None]
    ptx = e(b1x) * px3 + e(b2x) * py3 + e(b3x) * pz3 + tx[:, :, None]
    pty = e(b1y) * px3 + e(b2y) * py3 + e(b3y) * pz3 + ty[:, :, None]
    ptz = e(b1z) * px3 + e(b2z) * py3 + e(b3z) * pz3 + tz[:, :, None]
    pp = ptx * ptx + pty * pty + ptz * ptz
    pcn = jnp.sqrt(pp[:, 0, :])
    big = jnp.dot(pcn, wpcn[...], preferred_element_type=F32)
    mcc = jnp.dot(mc[0], wmc[...], preferred_element_type=F32)
    bptx, bpty, bptz = bf(ptx), bf(pty), bf(ptz)

    def one_hand(jx_r, jy_r, jz_r, xh_r, ws_r, bias_r, out_r, off):
        jx, jy, jz = jx_r[0], jy_r[0], jz_r[0]
        jj = (jx * jx + jy * jy + jz * jz)[:, :, None]
        jx3, jy3, jz3 = jx[:, :, None], jy[:, :, None], jz[:, :, None]
        s3 = bf(jx3) * bptx + bf(jy3) * bpty + bf(jz3) * bptz
        d2 = jj + pp - 2.0 * s3
        idx = jnp.argmin(d2, axis=2)
        oh = (jax.lax.broadcasted_iota(jnp.int32, (L, J, N), 2)
              == idx[:, :, None]).astype(F32)
        cx = jnp.sum(oh * ptx, axis=2)
        cy = jnp.sum(oh * pty, axis=2)
        cz = jnp.sum(oh * ptz, axis=2)
        ax = jnp.exp(-50.0 * (jx - cx) ** 2)
        ay = jnp.exp(-50.0 * (jy - cy) ** 2)
        az = jnp.exp(-50.0 * (jz - cz) ** 2)
        small = jnp.concatenate([xh_r[0], jx, jy, jz, ax, ay, az], axis=1)
        res = (jnp.dot(small, ws_r[...], preferred_element_type=F32)
               + big[:, off:off + HID] + mcc[:, off:off + HID] + bias_r[...])
        out_r[0] = res

    one_hand(jxl, jyl, jzl, xhl, wsl, bl, ol, 0)
    one_hand(jxr, jyr, jzr, xhr, wsr, br, orr, HID)


def _ln(x, g, b):
    mu = jnp.mean(x, axis=-1, keepdims=True)
    xc = x - mu
    var = jnp.mean(xc * xc, axis=-1, keepdims=True)
    return xc * jax.lax.rsqrt(var + 1e-5) * g + b


def _layer_body(x_ref, wqkv, bqkv, wo, bo_, w1, b1_, w2, b2_, g1, be1, g2, be2,
                out_ref, xs):
    l = pl.program_id(0)
    b = pl.program_id(1)

    @pl.when(l == 0)
    def _():
        xs[b] = x_ref[0]

    x = xs[b]
    qkv = jnp.dot(x, wqkv[0], preferred_element_type=F32) + bqkv[0]
    scale = 1.0 / np.sqrt(DH)
    outs = []
    for h in range(NH):
        q = qkv[:, h * DH:(h + 1) * DH]
        k = qkv[:, HID + h * DH:HID + (h + 1) * DH]
        v = qkv[:, 2 * HID + h * DH:2 * HID + (h + 1) * DH]
        s = jax.lax.dot_general(q, k, (((1,), (1,)), ((), ())),
                                preferred_element_type=F32) * scale
        m = jnp.max(s, axis=-1, keepdims=True)
        e = jnp.exp(s - m)
        p = e / jnp.sum(e, axis=-1, keepdims=True)
        outs.append(jnp.dot(p, v, preferred_element_type=F32))
    o = jnp.concatenate(outs, axis=1)
    attn = jnp.dot(o, wo[0], preferred_element_type=F32) + bo_[0]
    h1 = _ln(x + attn, g1[0], be1[0])
    ff = jnp.dot(jnp.maximum(jnp.dot(h1, w1[0], preferred_element_type=F32) + b1_[0], 0.0),
                 w2[0], preferred_element_type=F32) + b2_[0]
    new = _ln(h1 + ff, g2[0], be2[0])
    xs[b] = new
    out_ref[0] = new


def _proj_body(xe, xo, wl, bl, wr, br, ol, orr):
    ol[...] = jnp.dot(xe[...], wl[...], preferred_element_type=F32) + bl[...]
    orr[...] = jnp.dot(xo[...], wr[...], preferred_element_type=F32) + br[...]


def kernel(x_lhand, x_rhand, j_lhand, j_rhand, m_contact, x_obj, point_cloud,
           fc_lw, fc_lb, fc_rw, fc_rb, out_lw, out_lb, out_rw, out_rb,
           Wqkv, bqkv, Wo, bo, W1, b1f, W2, b2f, ln1_g, ln1_b, ln2_g, ln2_b):
    jxl, jyl, jzl = j_lhand[..., 0], j_lhand[..., 1], j_lhand[..., 2]
    jxr, jyr, jzr = j_rhand[..., 0], j_rhand[..., 1], j_rhand[..., 2]
    px = point_cloud[:, :, 0].reshape(B, 1, N)
    py = point_cloud[:, :, 1].reshape(B, 1, N)
    pz = point_cloud[:, :, 2].reshape(B, 1, N)
    mc3 = m_contact.reshape(B, 1, N)

    def splitw(W):
        wxh, wj = W[0:HAND], W[HAND:HAND + 3 * J]
        wmc_ = W[162:162 + N]
        wpcn_ = W[1186:1186 + N]
        watt = W[2210:2210 + 3 * J]
        small = jnp.concatenate([wxh, wj[0::3], wj[1::3], wj[2::3],
                                 watt[0::3], watt[1::3], watt[2::3]], axis=0)
        return small, wmc_, wpcn_

    wsl, wmcl, wpcnl = splitw(fc_lw)
    wsr, wmcr, wpcnr = splitw(fc_rw)
    wmc = jnp.concatenate([wmcl, wmcr], axis=1)
    wpcn = jnp.concatenate([wpcnl, wpcnr], axis=1)
    pe_f = _pe_table(L, HID)
    pe_a = _pe_table(2, HID)
    bl = jnp.asarray(pe_f + pe_a[0:1]) + fc_lb[None, :]
    br = jnp.asarray(pe_f + pe_a[1:2]) + fc_rb[None, :]

    bspec = lambda shp: pl.BlockSpec(shp, lambda i: (i,) + (0,) * (len(shp) - 1))
    cspec = lambda shp: pl.BlockSpec(shp, lambda i: (0,) * len(shp))
    xl, xr = pl.pallas_call(
        _geom_body,
        grid=(B,),
        in_specs=[bspec((1, L, 10))]
        + [bspec((1, L, J))] * 6
        + [bspec((1, 1, N))] * 4
        + [bspec((1, L, HAND))] * 2
        + [cspec((HAND + 6 * J, HID))] * 2
        + [cspec((N, 2 * HID))] * 2
        + [cspec((L, HID))] * 2,
        out_specs=[bspec((1, L, HID))] * 2,
        out_shape=[jax.ShapeDtypeStruct((B, L, HID), F32)] * 2,
        compiler_params=pltpu.CompilerParams(
            dimension_semantics=("arbitrary",),
            vmem_limit_bytes=120 * 1024 * 1024,
        ),
    )(x_obj, jxl, jyl, jzl, jxr, jyr, jzr, px, py, pz, mc3, x_lhand, x_rhand,
      wsl, wsr, wmc, wpcn, bl, br)

    x = jnp.stack([xl, xr], axis=2).reshape(B, 2 * L, HID)

    xspec = pl.BlockSpec((1, 2 * L, HID), lambda l, b: (b, 0, 0))
    lspec = lambda shp: pl.BlockSpec((1,) + shp, lambda l, b: (l, 0, 0))
    x = pl.pallas_call(
        _layer_body,
        grid=(NL, B),
        in_specs=[
            xspec,
            lspec((HID, 3 * HID)), lspec((1, 3 * HID)),
            lspec((HID, HID)), lspec((1, HID)),
            lspec((HID, 4 * HID)), lspec((1, 4 * HID)),
            lspec((4 * HID, HID)), lspec((1, HID)),
            lspec((1, HID)), lspec((1, HID)), lspec((1, HID)), lspec((1, HID)),
        ],
        out_specs=xspec,
        out_shape=jax.ShapeDtypeStruct((B, 2 * L, HID), F32),
        scratch_shapes=[pltpu.VMEM((B, 2 * L, HID), F32)],
        compiler_params=pltpu.CompilerParams(
            dimension_semantics=("arbitrary", "arbitrary"),
            vmem_limit_bytes=120 * 1024 * 1024,
        ),
    )(x, Wqkv, bqkv.reshape(NL, 1, 3 * HID), Wo, bo.reshape(NL, 1, HID),
      W1, b1f.reshape(NL, 1, 4 * HID), W2, b2f.reshape(NL, 1, HID),
      ln1_g.reshape(NL, 1, HID), ln1_b.reshape(NL, 1, HID),
      ln2_g.reshape(NL, 1, HID), ln2_b.reshape(NL, 1, HID))

    x4 = x.reshape(B, L, 2, HID)
    xe = x4[:, :, 0, :].reshape(B * L, HID)
    xo = x4[:, :, 1, :].reshape(B * L, HID)
    wlp = jnp.pad(out_lw, ((0, 0), (0, 128 - HAND)))
    wrp = jnp.pad(out_rw, ((0, 0), (0, 128 - HAND)))
    blp = jnp.pad(out_lb, (0, 128 - HAND)).reshape(1, 128)
    brp = jnp.pad(out_rb, (0, 128 - HAND)).reshape(1, 128)
    cs = lambda shp: pl.BlockSpec(shp, lambda: (0,) * len(shp))
    ol, orr = pl.pallas_call(
        _proj_body,
        in_specs=[cs((B * L, HID)), cs((B * L, HID)),
                  cs((HID, 128)), cs((1, 128)), cs((HID, 128)), cs((1, 128))],
        out_specs=[cs((B * L, 128))] * 2,
        out_shape=[jax.ShapeDtypeStruct((B * L, 128), F32)] * 2,
        compiler_params=pltpu.CompilerParams(
            vmem_limit_bytes=120 * 1024 * 1024,
        ),
    )(xe, xo, wlp, blp, wrp, brp)
    out_l = ol.reshape(B, L, 128)[:, :, :HAND]
    out_r = orr.reshape(B, L, 128)[:, :, :HAND]
    return out_l, out_r


# speedup vs baseline: 1.1466x; 1.1229x over previous
import numpy as np
import jax
import jax.numpy as jnp
from jax.experimental import pallas as pl
from jax.experimental.pallas import tpu as pltpu

B, L, N = 16, 64, 1024
HAND, J, HID = 99, 21, 512
NL, NH, DH = 4, 8, 64
F32 = jnp.float32


def _pe_table(n, d):
    pos = np.arange(n)[:, None].astype(np.float32)
    div = np.exp(np.arange(0, d, 2).astype(np.float32) * (-np.log(10000.0) / d))
    pe = np.zeros((n, d), np.float32)
    pe[:, 0::2] = np.sin(pos * div)
    pe[:, 1::2] = np.cos(pos * div)
    return pe


def _geom_body(xobj, jxl, jyl, jzl, jxr, jyr, jzr, px, py, pz, mc, xhl, xhr,
               wsl, wsr, wmc, wpcn, bl, br, ol, orr):
    xo = xobj[0]
    tx, ty, tz = xo[:, 0:1], xo[:, 1:2], xo[:, 2:3]
    a1x, a1y, a1z = xo[:, 3:4], xo[:, 4:5], xo[:, 5:6]
    a2x, a2y, a2z = xo[:, 6:7], xo[:, 7:8], xo[:, 8:9]
    n1 = jax.lax.rsqrt(a1x * a1x + a1y * a1y + a1z * a1z)
    b1x, b1y, b1z = a1x * n1, a1y * n1, a1z * n1
    dd = b1x * a2x + b1y * a2y + b1z * a2z
    c2x, c2y, c2z = a2x - dd * b1x, a2y - dd * b1y, a2z - dd * b1z
    n2 = jax.lax.rsqrt(c2x * c2x + c2y * c2y + c2z * c2z)
    b2x, b2y, b2z = c2x * n2, c2y * n2, c2z * n2
    b3x = b1y * b2z - b1z * b2y
    b3y = b1z * b2x - b1x * b2z
    b3z = b1x * b2y - b1y * b2x

    bf = lambda v: v.astype(jnp.bfloat16).astype(F32)
    px2, py2, pz2 = bf(px[0]), bf(py[0]), bf(pz[0])
    ptx2 = bf(b1x) * px2 + bf(b2x) * py2 + bf(b3x) * pz2 + tx
    pty2 = bf(b1y) * px2 + bf(b2y) * py2 + bf(b3y) * pz2 + ty
    ptz2 = bf(b1z) * px2 + bf(b2z) * py2 + bf(b3z) * pz2 + tz
    pp2 = ptx2 * ptx2 + pty2 * pty2 + ptz2 * ptz2
    pcn = jnp.sqrt(pp2)
    big = jnp.dot(pcn, wpcn[...], preferred_element_type=F32)
    mcc = jnp.dot(mc[0], wmc[...], preferred_element_type=F32)
    ptx, pty, ptz = ptx2[:, None, :], pty2[:, None, :], ptz2[:, None, :]
    pp = pp2[:, None, :]
    bptx, bpty, bptz = bf(ptx), bf(pty), bf(ptz)

    def one_hand(jx_r, jy_r, jz_r, xh_r, ws_r, bias_r, out_r, off):
        jx, jy, jz = jx_r[0], jy_r[0], jz_r[0]
        jj = (jx * jx + jy * jy + jz * jz)[:, :, None]
        jx3, jy3, jz3 = jx[:, :, None], jy[:, :, None], jz[:, :, None]
        s3 = bf(jx3) * bptx + bf(jy3) * bpty + bf(jz3) * bptz
        d2 = jj + pp - 2.0 * s3
        idx = jnp.argmin(d2, axis=2)
        oh = (jax.lax.broadcasted_iota(jnp.int32, (L, J, N), 2)
              == idx[:, :, None]).astype(F32)
        cx = jnp.sum(oh * ptx, axis=2)
        cy = jnp.sum(oh * pty, axis=2)
        cz = jnp.sum(oh * ptz, axis=2)
        ax = jnp.exp(-50.0 * (jx - cx) ** 2)
        ay = jnp.exp(-50.0 * (jy - cy) ** 2)
        az = jnp.exp(-50.0 * (jz - cz) ** 2)
        small = jnp.concatenate([xh_r[0], jx, jy, jz, ax, ay, az], axis=1)
        res = (jnp.dot(small, ws_r[...], preferred_element_type=F32)
               + big[:, off:off + HID] + mcc[:, off:off + HID] + bias_r[...])
        out_r[0] = res

    one_hand(jxl, jyl, jzl, xhl, wsl, bl, ol, 0)
    one_hand(jxr, jyr, jzr, xhr, wsr, br, orr, HID)


def _ln(x, g, b):
    mu = jnp.mean(x, axis=-1, keepdims=True)
    xc = x - mu
    var = jnp.mean(xc * xc, axis=-1, keepdims=True)
    return xc * jax.lax.rsqrt(var + 1e-5) * g + b


MB = 2


def _layer_body(x_ref, wqkv, bqkv, wo, bo_, w1, b1_, w2, b2_, g1, be1, g2, be2,
                out_ref, xs):
    l = pl.program_id(0)
    b = pl.program_id(1)

    @pl.when(l == 0)
    def _():
        xs[pl.ds(MB * b, MB)] = x_ref[...]

    x = xs[pl.ds(MB * b, MB)].reshape(MB * 2 * L, HID)
    qkv = jnp.dot(x, wqkv[0], preferred_element_type=F32) + bqkv[0]
    scale = 1.0 / np.sqrt(DH)
    rows = []
    for i in range(MB):
        r0 = i * 2 * L
        outs = []
        for h in range(NH):
            q = qkv[r0:r0 + 2 * L, h * DH:(h + 1) * DH]
            k = qkv[r0:r0 + 2 * L, HID + h * DH:HID + (h + 1) * DH]
            v = qkv[r0:r0 + 2 * L, 2 * HID + h * DH:2 * HID + (h + 1) * DH]
            sc = jax.lax.dot_general(q, k, (((1,), (1,)), ((), ())),
                                     preferred_element_type=F32) * scale
            m = jnp.max(sc, axis=-1, keepdims=True)
            e = jnp.exp(sc - m)
            p = e / jnp.sum(e, axis=-1, keepdims=True)
            outs.append(jnp.dot(p, v, preferred_element_type=F32))
        rows.append(jnp.concatenate(outs, axis=1))
    o = jnp.concatenate(rows, axis=0)
    attn = jnp.dot(o, wo[0], preferred_element_type=F32) + bo_[0]
    h1 = _ln(x + attn, g1[0], be1[0])
    ff = jnp.dot(jnp.maximum(jnp.dot(h1, w1[0], preferred_element_type=F32) + b1_[0], 0.0),
                 w2[0], preferred_element_type=F32) + b2_[0]
    new = _ln(h1 + ff, g2[0], be2[0])
    new3 = new.reshape(MB, 2 * L, HID)
    xs[pl.ds(MB * b, MB)] = new3
    out_ref[...] = new3


def _proj_body(xe, xo, wl, bl, wr, br, ol, orr):
    ol[...] = jnp.dot(xe[...], wl[...], preferred_element_type=F32) + bl[...]
    orr[...] = jnp.dot(xo[...], wr[...], preferred_element_type=F32) + br[...]


def kernel(x_lhand, x_rhand, j_lhand, j_rhand, m_contact, x_obj, point_cloud,
           fc_lw, fc_lb, fc_rw, fc_rb, out_lw, out_lb, out_rw, out_rb,
           Wqkv, bqkv, Wo, bo, W1, b1f, W2, b2f, ln1_g, ln1_b, ln2_g, ln2_b):
    jxl, jyl, jzl = j_lhand[..., 0], j_lhand[..., 1], j_lhand[..., 2]
    jxr, jyr, jzr = j_rhand[..., 0], j_rhand[..., 1], j_rhand[..., 2]
    px = point_cloud[:, :, 0].reshape(B, 1, N)
    py = point_cloud[:, :, 1].reshape(B, 1, N)
    pz = point_cloud[:, :, 2].reshape(B, 1, N)
    mc3 = m_contact.reshape(B, 1, N)

    def splitw(W):
        wxh, wj = W[0:HAND], W[HAND:HAND + 3 * J]
        wmc_ = W[162:162 + N]
        wpcn_ = W[1186:1186 + N]
        watt = W[2210:2210 + 3 * J]
        small = jnp.concatenate([wxh, wj[0::3], wj[1::3], wj[2::3],
                                 watt[0::3], watt[1::3], watt[2::3]], axis=0)
        return small, wmc_, wpcn_

    wsl, wmcl, wpcnl = splitw(fc_lw)
    wsr, wmcr, wpcnr = splitw(fc_rw)
    wmc = jnp.concatenate([wmcl, wmcr], axis=1)
    wpcn = jnp.concatenate([wpcnl, wpcnr], axis=1)
    pe_f = _pe_table(L, HID)
    pe_a = _pe_table(2, HID)
    bl = jnp.asarray(pe_f + pe_a[0:1]) + fc_lb[None, :]
    br = jnp.asarray(pe_f + pe_a[1:2]) + fc_rb[None, :]

    bspec = lambda shp: pl.BlockSpec(shp, lambda i: (i,) + (0,) * (len(shp) - 1))
    cspec = lambda shp: pl.BlockSpec(shp, lambda i: (0,) * len(shp))
    xl, xr = pl.pallas_call(
        _geom_body,
        grid=(B,),
        in_specs=[bspec((1, L, 10))]
        + [bspec((1, L, J))] * 6
        + [bspec((1, 1, N))] * 4
        + [bspec((1, L, HAND))] * 2
        + [cspec((HAND + 6 * J, HID))] * 2
        + [cspec((N, 2 * HID))] * 2
        + [cspec((L, HID))] * 2,
        out_specs=[bspec((1, L, HID))] * 2,
        out_shape=[jax.ShapeDtypeStruct((B, L, HID), F32)] * 2,
        compiler_params=pltpu.CompilerParams(
            dimension_semantics=("arbitrary",),
            vmem_limit_bytes=120 * 1024 * 1024,
        ),
    )(x_obj, jxl, jyl, jzl, jxr, jyr, jzr, px, py, pz, mc3, x_lhand, x_rhand,
      wsl, wsr, wmc, wpcn, bl, br)

    x = jnp.stack([xl, xr], axis=2).reshape(B, 2 * L, HID)

    xspec = pl.BlockSpec((MB, 2 * L, HID), lambda l, b: (b, 0, 0))
    lspec = lambda shp: pl.BlockSpec((1,) + shp, lambda l, b: (l, 0, 0))
    x = pl.pallas_call(
        _layer_body,
        grid=(NL, B // MB),
        in_specs=[
            xspec,
            lspec((HID, 3 * HID)), lspec((1, 3 * HID)),
            lspec((HID, HID)), lspec((1, HID)),
            lspec((HID, 4 * HID)), lspec((1, 4 * HID)),
            lspec((4 * HID, HID)), lspec((1, HID)),
            lspec((1, HID)), lspec((1, HID)), lspec((1, HID)), lspec((1, HID)),
        ],
        out_specs=xspec,
        out_shape=jax.ShapeDtypeStruct((B, 2 * L, HID), F32),
        scratch_shapes=[pltpu.VMEM((B, 2 * L, HID), F32)],
        compiler_params=pltpu.CompilerParams(
            dimension_semantics=("arbitrary", "arbitrary"),
            vmem_limit_bytes=120 * 1024 * 1024,
        ),
    )(x, Wqkv, bqkv.reshape(NL, 1, 3 * HID), Wo, bo.reshape(NL, 1, HID),
      W1, b1f.reshape(NL, 1, 4 * HID), W2, b2f.reshape(NL, 1, HID),
      ln1_g.reshape(NL, 1, HID), ln1_b.reshape(NL, 1, HID),
      ln2_g.reshape(NL, 1, HID), ln2_b.reshape(NL, 1, HID))

    x4 = x.reshape(B, L, 2, HID)
    xe = x4[:, :, 0, :].reshape(B * L, HID)
    xo = x4[:, :, 1, :].reshape(B * L, HID)
    wlp = jnp.pad(out_lw, ((0, 0), (0, 128 - HAND)))
    wrp = jnp.pad(out_rw, ((0, 0), (0, 128 - HAND)))
    blp = jnp.pad(out_lb, (0, 128 - HAND)).reshape(1, 128)
    brp = jnp.pad(out_rb, (0, 128 - HAND)).reshape(1, 128)
    cs = lambda shp: pl.BlockSpec(shp, lambda: (0,) * len(shp))
    ol, orr = pl.pallas_call(
        _proj_body,
        in_specs=[cs((B * L, HID)), cs((B * L, HID)),
                  cs((HID, 128)), cs((1, 128)), cs((HID, 128)), cs((1, 128))],
        out_specs=[cs((B * L, 128))] * 2,
        out_shape=[jax.ShapeDtypeStruct((B * L, 128), F32)] * 2,
        compiler_params=pltpu.CompilerParams(
            vmem_limit_bytes=120 * 1024 * 1024,
        ),
    )(xe, xo, wlp, blp, wrp, brp)
    out_l = ol.reshape(B, L, 128)[:, :, :HAND]
    out_r = orr.reshape(B, L, 128)[:, :, :HAND]
    return out_l, out_r


# speedup vs baseline: 1.1606x; 1.0122x over previous
import numpy as np
import jax
import jax.numpy as jnp
from jax.experimental import pallas as pl
from jax.experimental.pallas import tpu as pltpu

B, L, N = 16, 64, 1024
HAND, J, HID = 99, 21, 512
NL, NH, DH = 4, 8, 64
F32 = jnp.float32


def _pe_table(n, d):
    pos = np.arange(n)[:, None].astype(np.float32)
    div = np.exp(np.arange(0, d, 2).astype(np.float32) * (-np.log(10000.0) / d))
    pe = np.zeros((n, d), np.float32)
    pe[:, 0::2] = np.sin(pos * div)
    pe[:, 1::2] = np.cos(pos * div)
    return pe


def _geom_body(xobj, jxl, jyl, jzl, jxr, jyr, jzr, px, py, pz, mc, xhl, xhr,
               wsl, wsr, wmc, wpcn, bl, br, ol, orr):
    xo = xobj[0]
    tx, ty, tz = xo[:, 0:1], xo[:, 1:2], xo[:, 2:3]
    a1x, a1y, a1z = xo[:, 3:4], xo[:, 4:5], xo[:, 5:6]
    a2x, a2y, a2z = xo[:, 6:7], xo[:, 7:8], xo[:, 8:9]
    n1 = jax.lax.rsqrt(a1x * a1x + a1y * a1y + a1z * a1z)
    b1x, b1y, b1z = a1x * n1, a1y * n1, a1z * n1
    dd = b1x * a2x + b1y * a2y + b1z * a2z
    c2x, c2y, c2z = a2x - dd * b1x, a2y - dd * b1y, a2z - dd * b1z
    n2 = jax.lax.rsqrt(c2x * c2x + c2y * c2y + c2z * c2z)
    b2x, b2y, b2z = c2x * n2, c2y * n2, c2z * n2
    b3x = b1y * b2z - b1z * b2y
    b3y = b1z * b2x - b1x * b2z
    b3z = b1x * b2y - b1y * b2x

    bf = lambda v: v.astype(jnp.bfloat16).astype(F32)
    px2, py2, pz2 = bf(px[0]), bf(py[0]), bf(pz[0])
    ptx2 = bf(b1x) * px2 + bf(b2x) * py2 + bf(b3x) * pz2 + tx
    pty2 = bf(b1y) * px2 + bf(b2y) * py2 + bf(b3y) * pz2 + ty
    ptz2 = bf(b1z) * px2 + bf(b2z) * py2 + bf(b3z) * pz2 + tz
    pp2 = ptx2 * ptx2 + pty2 * pty2 + ptz2 * ptz2
    pcn = jnp.sqrt(pp2)
    big = jnp.dot(pcn, wpcn[...], preferred_element_type=F32)
    mcc = jnp.dot(mc[0], wmc[...], preferred_element_type=F32)
    ptx, pty, ptz = ptx2[:, None, :], pty2[:, None, :], ptz2[:, None, :]
    pp = pp2[:, None, :]
    bptx, bpty, bptz = bf(ptx), bf(pty), bf(ptz)

    def one_hand(jx_r, jy_r, jz_r, xh_r, ws_r, bias_r, out_r, off):
        jx, jy, jz = jx_r[0], jy_r[0], jz_r[0]
        jj = (jx * jx + jy * jy + jz * jz)[:, :, None]
        jx3, jy3, jz3 = jx[:, :, None], jy[:, :, None], jz[:, :, None]
        s3 = bf(jx3) * bptx + bf(jy3) * bpty + bf(jz3) * bptz
        d2 = jj + pp - 2.0 * s3
        idx = jnp.argmin(d2, axis=2)
        oh = (jax.lax.broadcasted_iota(jnp.int32, (L, J, N), 2)
              == idx[:, :, None]).astype(F32)
        cx = jnp.sum(oh * ptx, axis=2)
        cy = jnp.sum(oh * pty, axis=2)
        cz = jnp.sum(oh * ptz, axis=2)
        ax = jnp.exp(-50.0 * (jx - cx) ** 2)
        ay = jnp.exp(-50.0 * (jy - cy) ** 2)
        az = jnp.exp(-50.0 * (jz - cz) ** 2)
        small = jnp.concatenate([xh_r[0], jx, jy, jz, ax, ay, az], axis=1)
        res = (jnp.dot(small, ws_r[...], preferred_element_type=F32)
               + big[:, off:off + HID] + mcc[:, off:off + HID] + bias_r[...])
        out_r[0] = res

    one_hand(jxl, jyl, jzl, xhl, wsl, bl, ol, 0)
    one_hand(jxr, jyr, jzr, xhr, wsr, br, orr, HID)


def _ln(x, g, b):
    mu = jnp.mean(x, axis=-1, keepdims=True)
    xc = x - mu
    var = jnp.mean(xc * xc, axis=-1, keepdims=True)
    return xc * jax.lax.rsqrt(var + 1e-5) * g + b


MB = 4


def _layer_body(x_ref, wqkv, bqkv, wo, bo_, w1, b1_, w2, b2_, g1, be1, g2, be2,
                out_ref, xs):
    l = pl.program_id(0)
    b = pl.program_id(1)

    @pl.when(l == 0)
    def _():
        xs[pl.ds(MB * b, MB)] = x_ref[...]

    x = xs[pl.ds(MB * b, MB)].reshape(MB * 2 * L, HID)
    qkv = jnp.dot(x, wqkv[0], preferred_element_type=F32) + bqkv[0]
    scale = 1.0 / np.sqrt(DH)
    rows = []
    for i in range(MB):
        r0 = i * 2 * L
        outs = []
        for h in range(NH):
            q = qkv[r0:r0 + 2 * L, h * DH:(h + 1) * DH]
            k = qkv[r0:r0 + 2 * L, HID + h * DH:HID + (h + 1) * DH]
            v = qkv[r0:r0 + 2 * L, 2 * HID + h * DH:2 * HID + (h + 1) * DH]
            sc = jax.lax.dot_general(q, k, (((1,), (1,)), ((), ())),
                                     preferred_element_type=F32) * scale
            m = jnp.max(sc, axis=-1, keepdims=True)
            e = jnp.exp(sc - m)
            p = e / jnp.sum(e, axis=-1, keepdims=True)
            outs.append(jnp.dot(p, v, preferred_element_type=F32))
        rows.append(jnp.concatenate(outs, axis=1))
    o = jnp.concatenate(rows, axis=0)
    attn = jnp.dot(o, wo[0], preferred_element_type=F32) + bo_[0]
    h1 = _ln(x + attn, g1[0], be1[0])
    ff = jnp.dot(jnp.maximum(jnp.dot(h1, w1[0], preferred_element_type=F32) + b1_[0], 0.0),
                 w2[0], preferred_element_type=F32) + b2_[0]
    new = _ln(h1 + ff, g2[0], be2[0])
    new3 = new.reshape(MB, 2 * L, HID)
    xs[pl.ds(MB * b, MB)] = new3
    out_ref[...] = new3


def _proj_body(xe, xo, wl, bl, wr, br, ol, orr):
    ol[...] = jnp.dot(xe[...], wl[...], preferred_element_type=F32) + bl[...]
    orr[...] = jnp.dot(xo[...], wr[...], preferred_element_type=F32) + br[...]


def kernel(x_lhand, x_rhand, j_lhand, j_rhand, m_contact, x_obj, point_cloud,
           fc_lw, fc_lb, fc_rw, fc_rb, out_lw, out_lb, out_rw, out_rb,
           Wqkv, bqkv, Wo, bo, W1, b1f, W2, b2f, ln1_g, ln1_b, ln2_g, ln2_b):
    jxl, jyl, jzl = j_lhand[..., 0], j_lhand[..., 1], j_lhand[..., 2]
    jxr, jyr, jzr = j_rhand[..., 0], j_rhand[..., 1], j_rhand[..., 2]
    px = point_cloud[:, :, 0].reshape(B, 1, N)
    py = point_cloud[:, :, 1].reshape(B, 1, N)
    pz = point_cloud[:, :, 2].reshape(B, 1, N)
    mc3 = m_contact.reshape(B, 1, N)

    def splitw(W):
        wxh, wj = W[0:HAND], W[HAND:HAND + 3 * J]
        wmc_ = W[162:162 + N]
        wpcn_ = W[1186:1186 + N]
        watt = W[2210:2210 + 3 * J]
        small = jnp.concatenate([wxh, wj[0::3], wj[1::3], wj[2::3],
                                 watt[0::3], watt[1::3], watt[2::3]], axis=0)
        return small, wmc_, wpcn_

    wsl, wmcl, wpcnl = splitw(fc_lw)
    wsr, wmcr, wpcnr = splitw(fc_rw)
    wmc = jnp.concatenate([wmcl, wmcr], axis=1)
    wpcn = jnp.concatenate([wpcnl, wpcnr], axis=1)
    pe_f = _pe_table(L, HID)
    pe_a = _pe_table(2, HID)
    bl = jnp.asarray(pe_f + pe_a[0:1]) + fc_lb[None, :]
    br = jnp.asarray(pe_f + pe_a[1:2]) + fc_rb[None, :]

    bspec = lambda shp: pl.BlockSpec(shp, lambda i: (i,) + (0,) * (len(shp) - 1))
    cspec = lambda shp: pl.BlockSpec(shp, lambda i: (0,) * len(shp))
    xl, xr = pl.pallas_call(
        _geom_body,
        grid=(B,),
        in_specs=[bspec((1, L, 10))]
        + [bspec((1, L, J))] * 6
        + [bspec((1, 1, N))] * 4
        + [bspec((1, L, HAND))] * 2
        + [cspec((HAND + 6 * J, HID))] * 2
        + [cspec((N, 2 * HID))] * 2
        + [cspec((L, HID))] * 2,
        out_specs=[bspec((1, L, HID))] * 2,
        out_shape=[jax.ShapeDtypeStruct((B, L, HID), F32)] * 2,
        compiler_params=pltpu.CompilerParams(
            dimension_semantics=("arbitrary",),
            vmem_limit_bytes=120 * 1024 * 1024,
        ),
    )(x_obj, jxl, jyl, jzl, jxr, jyr, jzr, px, py, pz, mc3, x_lhand, x_rhand,
      wsl, wsr, wmc, wpcn, bl, br)

    x = jnp.stack([xl, xr], axis=2).reshape(B, 2 * L, HID)

    xspec = pl.BlockSpec((MB, 2 * L, HID), lambda l, b: (b, 0, 0))
    lspec = lambda shp: pl.BlockSpec((1,) + shp, lambda l, b: (l, 0, 0))
    x = pl.pallas_call(
        _layer_body,
        grid=(NL, B // MB),
        in_specs=[
            xspec,
            lspec((HID, 3 * HID)), lspec((1, 3 * HID)),
            lspec((HID, HID)), lspec((1, HID)),
            lspec((HID, 4 * HID)), lspec((1, 4 * HID)),
            lspec((4 * HID, HID)), lspec((1, HID)),
            lspec((1, HID)), lspec((1, HID)), lspec((1, HID)), lspec((1, HID)),
        ],
        out_specs=xspec,
        out_shape=jax.ShapeDtypeStruct((B, 2 * L, HID), F32),
        scratch_shapes=[pltpu.VMEM((B, 2 * L, HID), F32)],
        compiler_params=pltpu.CompilerParams(
            dimension_semantics=("arbitrary", "arbitrary"),
            vmem_limit_bytes=120 * 1024 * 1024,
        ),
    )(x, Wqkv, bqkv.reshape(NL, 1, 3 * HID), Wo, bo.reshape(NL, 1, HID),
      W1, b1f.reshape(NL, 1, 4 * HID), W2, b2f.reshape(NL, 1, HID),
      ln1_g.reshape(NL, 1, HID), ln1_b.reshape(NL, 1, HID),
      ln2_g.reshape(NL, 1, HID), ln2_b.reshape(NL, 1, HID))

    x4 = x.reshape(B, L, 2, HID)
    xe = x4[:, :, 0, :].reshape(B * L, HID)
    xo = x4[:, :, 1, :].reshape(B * L, HID)
    wlp = jnp.pad(out_lw, ((0, 0), (0, 128 - HAND)))
    wrp = jnp.pad(out_rw, ((0, 0), (0, 128 - HAND)))
    blp = jnp.pad(out_lb, (0, 128 - HAND)).reshape(1, 128)
    brp = jnp.pad(out_rb, (0, 128 - HAND)).reshape(1, 128)
    cs = lambda shp: pl.BlockSpec(shp, lambda: (0,) * len(shp))
    ol, orr = pl.pallas_call(
        _proj_body,
        in_specs=[cs((B * L, HID)), cs((B * L, HID)),
                  cs((HID, 128)), cs((1, 128)), cs((HID, 128)), cs((1, 128))],
        out_specs=[cs((B * L, 128))] * 2,
        out_shape=[jax.ShapeDtypeStruct((B * L, 128), F32)] * 2,
        compiler_params=pltpu.CompilerParams(
            vmem_limit_bytes=120 * 1024 * 1024,
        ),
    )(xe, xo, wlp, blp, wrp, brp)
    out_l = ol.reshape(B, L, 128)[:, :, :HAND]
    out_r = orr.reshape(B, L, 128)[:, :, :HAND]
    return out_l, out_r


# speedup vs baseline: 1.5009x; 1.2931x over previous
import numpy as np
import jax
import jax.numpy as jnp
from jax.experimental import pallas as pl
from jax.experimental.pallas import tpu as pltpu

B, L, N = 16, 64, 1024
HAND, J, HID = 99, 21, 512
NL, NH, DH = 4, 8, 64
F32 = jnp.float32


def _pe_table(n, d):
    pos = np.arange(n)[:, None].astype(np.float32)
    div = np.exp(np.arange(0, d, 2).astype(np.float32) * (-np.log(10000.0) / d))
    pe = np.zeros((n, d), np.float32)
    pe[:, 0::2] = np.sin(pos * div)
    pe[:, 1::2] = np.cos(pos * div)
    return pe


def _geom_body(xobj, jxl, jyl, jzl, jxr, jyr, jzr, px, py, pz, mc, xhl, xhr,
               wsl, wsr, wmc, wpcn, bl, br, ol, orr):
    xo = xobj[0]
    tx, ty, tz = xo[:, 0:1], xo[:, 1:2], xo[:, 2:3]
    a1x, a1y, a1z = xo[:, 3:4], xo[:, 4:5], xo[:, 5:6]
    a2x, a2y, a2z = xo[:, 6:7], xo[:, 7:8], xo[:, 8:9]
    n1 = jax.lax.rsqrt(a1x * a1x + a1y * a1y + a1z * a1z)
    b1x, b1y, b1z = a1x * n1, a1y * n1, a1z * n1
    dd = b1x * a2x + b1y * a2y + b1z * a2z
    c2x, c2y, c2z = a2x - dd * b1x, a2y - dd * b1y, a2z - dd * b1z
    n2 = jax.lax.rsqrt(c2x * c2x + c2y * c2y + c2z * c2z)
    b2x, b2y, b2z = c2x * n2, c2y * n2, c2z * n2
    b3x = b1y * b2z - b1z * b2y
    b3y = b1z * b2x - b1x * b2z
    b3z = b1x * b2y - b1y * b2x

    bf = lambda v: v.astype(jnp.bfloat16).astype(F32)
    px2, py2, pz2 = bf(px[0]), bf(py[0]), bf(pz[0])
    ptx2 = bf(b1x) * px2 + bf(b2x) * py2 + bf(b3x) * pz2 + tx
    pty2 = bf(b1y) * px2 + bf(b2y) * py2 + bf(b3y) * pz2 + ty
    ptz2 = bf(b1z) * px2 + bf(b2z) * py2 + bf(b3z) * pz2 + tz
    pp2 = ptx2 * ptx2 + pty2 * pty2 + ptz2 * ptz2
    pcn = jnp.sqrt(pp2)
    big = jnp.dot(pcn, wpcn[...], preferred_element_type=F32)
    mcc = jnp.dot(mc[0], wmc[...], preferred_element_type=F32)
    ptx, pty, ptz = ptx2[:, None, :], pty2[:, None, :], ptz2[:, None, :]
    pp = pp2[:, None, :]
    bptx, bpty, bptz = bf(ptx), bf(pty), bf(ptz)

    def one_hand(jx_r, jy_r, jz_r, xh_r, ws_r, bias_r, out_r, off):
        jx, jy, jz = jx_r[0], jy_r[0], jz_r[0]
        jj = (jx * jx + jy * jy + jz * jz)[:, :, None]
        jx3, jy3, jz3 = jx[:, :, None], jy[:, :, None], jz[:, :, None]
        s3 = bf(jx3) * bptx + bf(jy3) * bpty + bf(jz3) * bptz
        d2 = jj + pp - 2.0 * s3
        idx = jnp.argmin(d2, axis=2)
        oh = (jax.lax.broadcasted_iota(jnp.int32, (L, J, N), 2)
              == idx[:, :, None]).astype(F32)
        cx = jnp.sum(oh * ptx, axis=2)
        cy = jnp.sum(oh * pty, axis=2)
        cz = jnp.sum(oh * ptz, axis=2)
        ax = jnp.exp(-50.0 * (jx - cx) ** 2)
        ay = jnp.exp(-50.0 * (jy - cy) ** 2)
        az = jnp.exp(-50.0 * (jz - cz) ** 2)
        small = jnp.concatenate([xh_r[0], jx, jy, jz, ax, ay, az], axis=1)
        res = (jnp.dot(small, ws_r[...], preferred_element_type=F32)
               + big[:, off:off + HID] + mcc[:, off:off + HID] + bias_r[...])
        out_r[0] = res

    one_hand(jxl, jyl, jzl, xhl, wsl, bl, ol, 0)
    one_hand(jxr, jyr, jzr, xhr, wsr, br, orr, HID)


def _ln(x, g, b):
    mu = jnp.mean(x, axis=-1, keepdims=True)
    xc = x - mu
    var = jnp.mean(xc * xc, axis=-1, keepdims=True)
    return xc * jax.lax.rsqrt(var + 1e-5) * g + b


MB = 4


def _layer_body(x_ref, wqkv, bqkv, wo, bo_, w1, b1_, w2, b2_, g1, be1, g2, be2,
                out_ref, xs):
    l = pl.program_id(0)
    b = pl.program_id(1)

    @pl.when(l == 0)
    def _():
        xs[pl.ds(MB * b, MB)] = x_ref[...]

    x = xs[pl.ds(MB * b, MB)].reshape(MB * 2 * L, HID)
    qkv = jnp.dot(x, wqkv[0], preferred_element_type=F32) + bqkv[0]
    scale = 1.0 / np.sqrt(DH)
    scs = []
    for i in range(MB):
        r0 = i * 2 * L
        for h in range(NH):
            q = qkv[r0:r0 + 2 * L, h * DH:(h + 1) * DH]
            k = qkv[r0:r0 + 2 * L, HID + h * DH:HID + (h + 1) * DH]
            scs.append(jax.lax.dot_general(q, k, (((1,), (1,)), ((), ())),
                                           preferred_element_type=F32) * scale)
    ps = []
    for sc in scs:
        m = jnp.max(sc, axis=-1, keepdims=True)
        e = jnp.exp(sc - m)
        ps.append(e / jnp.sum(e, axis=-1, keepdims=True))
    rows = []
    for i in range(MB):
        r0 = i * 2 * L
        outs = []
        for h in range(NH):
            v = qkv[r0:r0 + 2 * L, 2 * HID + h * DH:2 * HID + (h + 1) * DH]
            outs.append(jnp.dot(ps[i * NH + h], v, preferred_element_type=F32))
        rows.append(jnp.concatenate(outs, axis=1))
    o = jnp.concatenate(rows, axis=0)
    attn = jnp.dot(o, wo[0], preferred_element_type=F32) + bo_[0]
    h1 = _ln(x + attn, g1[0], be1[0])
    ff = jnp.dot(jnp.maximum(jnp.dot(h1, w1[0], preferred_element_type=F32) + b1_[0], 0.0),
                 w2[0], preferred_element_type=F32) + b2_[0]
    new = _ln(h1 + ff, g2[0], be2[0])
    new3 = new.reshape(MB, 2 * L, HID)
    xs[pl.ds(MB * b, MB)] = new3
    out_ref[...] = new3


def _proj_body(xe, xo, wl, bl, wr, br, ol, orr):
    ol[...] = jnp.dot(xe[...], wl[...], preferred_element_type=F32) + bl[...]
    orr[...] = jnp.dot(xo[...], wr[...], preferred_element_type=F32) + br[...]


def kernel(x_lhand, x_rhand, j_lhand, j_rhand, m_contact, x_obj, point_cloud,
           fc_lw, fc_lb, fc_rw, fc_rb, out_lw, out_lb, out_rw, out_rb,
           Wqkv, bqkv, Wo, bo, W1, b1f, W2, b2f, ln1_g, ln1_b, ln2_g, ln2_b):
    jxl, jyl, jzl = j_lhand[..., 0], j_lhand[..., 1], j_lhand[..., 2]
    jxr, jyr, jzr = j_rhand[..., 0], j_rhand[..., 1], j_rhand[..., 2]
    px = point_cloud[:, :, 0].reshape(B, 1, N)
    py = point_cloud[:, :, 1].reshape(B, 1, N)
    pz = point_cloud[:, :, 2].reshape(B, 1, N)
    mc3 = m_contact.reshape(B, 1, N)

    def splitw(W):
        wxh, wj = W[0:HAND], W[HAND:HAND + 3 * J]
        wmc_ = W[162:162 + N]
        wpcn_ = W[1186:1186 + N]
        watt = W[2210:2210 + 3 * J]
        small = jnp.concatenate([wxh, wj[0::3], wj[1::3], wj[2::3],
                                 watt[0::3], watt[1::3], watt[2::3]], axis=0)
        return small, wmc_, wpcn_

    wsl, wmcl, wpcnl = splitw(fc_lw)
    wsr, wmcr, wpcnr = splitw(fc_rw)
    wmc = jnp.concatenate([wmcl, wmcr], axis=1)
    wpcn = jnp.concatenate([wpcnl, wpcnr], axis=1)
    pe_f = _pe_table(L, HID)
    pe_a = _pe_table(2, HID)
    bl = jnp.asarray(pe_f + pe_a[0:1]) + fc_lb[None, :]
    br = jnp.asarray(pe_f + pe_a[1:2]) + fc_rb[None, :]

    bspec = lambda shp: pl.BlockSpec(shp, lambda i: (i,) + (0,) * (len(shp) - 1))
    cspec = lambda shp: pl.BlockSpec(shp, lambda i: (0,) * len(shp))
    xl, xr = pl.pallas_call(
        _geom_body,
        grid=(B,),
        in_specs=[bspec((1, L, 10))]
        + [bspec((1, L, J))] * 6
        + [bspec((1, 1, N))] * 4
        + [bspec((1, L, HAND))] * 2
        + [cspec((HAND + 6 * J, HID))] * 2
        + [cspec((N, 2 * HID))] * 2
        + [cspec((L, HID))] * 2,
        out_specs=[bspec((1, L, HID))] * 2,
        out_shape=[jax.ShapeDtypeStruct((B, L, HID), F32)] * 2,
        compiler_params=pltpu.CompilerParams(
            dimension_semantics=("arbitrary",),
            vmem_limit_bytes=120 * 1024 * 1024,
        ),
    )(x_obj, jxl, jyl, jzl, jxr, jyr, jzr, px, py, pz, mc3, x_lhand, x_rhand,
      wsl, wsr, wmc, wpcn, bl, br)

    x = jnp.stack([xl, xr], axis=2).reshape(B, 2 * L, HID)

    xspec = pl.BlockSpec((MB, 2 * L, HID), lambda l, b: (b, 0, 0))
    lspec = lambda shp: pl.BlockSpec((1,) + shp, lambda l, b: (l, 0, 0))
    x = pl.pallas_call(
        _layer_body,
        grid=(NL, B // MB),
        in_specs=[
            xspec,
            lspec((HID, 3 * HID)), lspec((1, 3 * HID)),
            lspec((HID, HID)), lspec((1, HID)),
            lspec((HID, 4 * HID)), lspec((1, 4 * HID)),
            lspec((4 * HID, HID)), lspec((1, HID)),
            lspec((1, HID)), lspec((1, HID)), lspec((1, HID)), lspec((1, HID)),
        ],
        out_specs=xspec,
        out_shape=jax.ShapeDtypeStruct((B, 2 * L, HID), F32),
        scratch_shapes=[pltpu.VMEM((B, 2 * L, HID), F32)],
        compiler_params=pltpu.CompilerParams(
            dimension_semantics=("arbitrary", "arbitrary"),
            vmem_limit_bytes=120 * 1024 * 1024,
        ),
    )(x, Wqkv, bqkv.reshape(NL, 1, 3 * HID), Wo, bo.reshape(NL, 1, HID),
      W1, b1f.reshape(NL, 1, 4 * HID), W2, b2f.reshape(NL, 1, HID),
      ln1_g.reshape(NL, 1, HID), ln1_b.reshape(NL, 1, HID),
      ln2_g.reshape(NL, 1, HID), ln2_b.reshape(NL, 1, HID))

    x4 = x.reshape(B, L, 2, HID)
    xe = x4[:, :, 0, :].reshape(B * L, HID)
    xo = x4[:, :, 1, :].reshape(B * L, HID)
    wlp = jnp.pad(out_lw, ((0, 0), (0, 128 - HAND)))
    wrp = jnp.pad(out_rw, ((0, 0), (0, 128 - HAND)))
    blp = jnp.pad(out_lb, (0, 128 - HAND)).reshape(1, 128)
    brp = jnp.pad(out_rb, (0, 128 - HAND)).reshape(1, 128)
    cs = lambda shp: pl.BlockSpec(shp, lambda: (0,) * len(shp))
    ol, orr = pl.pallas_call(
        _proj_body,
        in_specs=[cs((B * L, HID)), cs((B * L, HID)),
                  cs((HID, 128)), cs((1, 128)), cs((HID, 128)), cs((1, 128))],
        out_specs=[cs((B * L, 128))] * 2,
        out_shape=[jax.ShapeDtypeStruct((B * L, 128), F32)] * 2,
        compiler_params=pltpu.CompilerParams(
            vmem_limit_bytes=120 * 1024 * 1024,
        ),
    )(xe, xo, wlp, blp, wrp, brp)
    out_l = ol.reshape(B, L, 128)[:, :, :HAND]
    out_r = orr.reshape(B, L, 128)[:, :, :HAND]
    return out_l, out_r
